# baseline (speedup 1.0000x reference)
import functools

import jax
import jax.numpy as jnp
from jax import lax
from jax.experimental import pallas as pl
from jax.experimental.pallas import tpu as pltpu
from jax.experimental.pallas import tpu_sc as plsc

NUM_EMBEDDINGS = 1000000
EMBEDDING_DIM = 16
BATCH = 16384

_NC = 2
_NS = 16
_NW = _NC * _NS
_BPW = BATCH // _NW
_CHUNK = 128
_NCHUNK = _BPW // _CHUNK


def _gather_body(table_hbm, idx_hbm, out_hbm, idx_v, rows_v, sem):
    wid = lax.axis_index("s") * _NC + lax.axis_index("c")
    pltpu.sync_copy(idx_hbm.at[pl.ds(wid * _NCHUNK, _NCHUNK)], idx_v)
    copies = []
    for j in range(_NCHUNK):
        copies.append(
            pltpu.async_copy(
                table_hbm.at[idx_v.at[j]],
                rows_v.at[pl.ds(j * _CHUNK, _CHUNK)],
                sem,
            )
        )
    for c in copies:
        c.wait()
    pltpu.sync_copy(rows_v, out_hbm.at[pl.ds(wid * _BPW, _BPW)])


_mesh = plsc.VectorSubcoreMesh(core_axis_name="c", subcore_axis_name="s")

_gather_call = functools.partial(
    pl.kernel,
    out_type=jax.ShapeDtypeStruct((BATCH, EMBEDDING_DIM), jnp.float32),
    mesh=_mesh,
    scratch_types=[
        pltpu.VMEM((_NCHUNK, _CHUNK), jnp.int32),
        pltpu.VMEM((_BPW, EMBEDDING_DIM), jnp.float32),
        pltpu.SemaphoreType.DMA,
    ],
    compiler_params=pltpu.CompilerParams(use_tc_tiling_on_sc=False),
)(_gather_body)


def kernel(indices, table):
    idx = jnp.asarray(indices, jnp.int32).reshape(BATCH // _CHUNK, _CHUNK)
    return _gather_call(table, idx)


# speedup vs baseline: 5.3759x; 5.3759x over previous
import functools

import jax
import jax.numpy as jnp
from jax import lax
from jax.experimental import pallas as pl
from jax.experimental.pallas import tpu as pltpu
from jax.experimental.pallas import tpu_sc as plsc

NUM_EMBEDDINGS = 1000000
EMBEDDING_DIM = 16
BATCH = 16384

_NC = 2
_NS = 16
_NW = _NC * _NS
_BPW = BATCH // _NW
_G = 16


def _gather_body(tt_hbm, idx_hbm, ot_hbm, idx_v, out_v, sem, *blks):
    wid = lax.axis_index("s") * _NC + lax.axis_index("c")
    pltpu.sync_copy(idx_hbm.at[pl.ds(wid * _BPW, _BPW)], idx_v)

    lanes = lax.iota(jnp.int32, 16)

    def group(gi, carry):
        b0 = gi * _G
        idx16 = idx_v[pl.ds(b0, _G)]
        copies = []
        for g in range(_G):
            idx = idx16[g]
            blk0 = pl.multiple_of((idx >> 7) << 7, 128)
            copies.append(
                pltpu.async_copy(
                    tt_hbm.at[:, pl.ds(blk0, 128)], blks[g], sem
                )
            )
        for g in range(_G):
            copies[g].wait()
            col = jnp.broadcast_to(idx16[g] & 127, (16,))
            vals = plsc.load_gather(blks[g], [lanes, col])
            pos = jnp.broadcast_to(b0 + g, (16,))
            plsc.store_scatter(out_v, [lanes, pos], vals)
        return carry

    lax.fori_loop(0, _BPW // _G, group, 0)

    pltpu.sync_copy(out_v, ot_hbm.at[:, pl.ds(wid * _BPW, _BPW)])


_mesh = plsc.VectorSubcoreMesh(core_axis_name="c", subcore_axis_name="s")

_gather_call = functools.partial(
    pl.kernel,
    out_type=jax.ShapeDtypeStruct((EMBEDDING_DIM, BATCH), jnp.float32),
    mesh=_mesh,
    scratch_types=[
        pltpu.VMEM((_BPW,), jnp.int32),
        pltpu.VMEM((EMBEDDING_DIM, _BPW), jnp.float32),
        pltpu.SemaphoreType.DMA,
    ] + [pltpu.VMEM((EMBEDDING_DIM, 128), jnp.float32) for _ in range(_G)],
    compiler_params=pltpu.CompilerParams(needs_layout_passes=False),
)(_gather_body)


def kernel(indices, table):
    idx = jnp.asarray(indices, jnp.int32)
    return _gather_call(table.T, idx).T


# speedup vs baseline: 5.6299x; 1.0472x over previous
import functools

import jax
import jax.numpy as jnp
from jax import lax
from jax.experimental import pallas as pl
from jax.experimental.pallas import tpu as pltpu
from jax.experimental.pallas import tpu_sc as plsc

NUM_EMBEDDINGS = 1000000
EMBEDDING_DIM = 16
BATCH = 16384

_NC = 2
_NS = 16
_NW = _NC * _NS
_BPW = BATCH // _NW
_G = 16
_NGRP = _BPW // _G


def _gather_body(tt_hbm, idx_hbm, ot_hbm, idx_v, out_v, sem_a, sem_b,
                 *blks):
    wid = lax.axis_index("s") * _NC + lax.axis_index("c")
    pltpu.sync_copy(idx_hbm.at[pl.ds(wid * _BPW, _BPW)], idx_v)

    lanes = lax.iota(jnp.int32, 16)
    blk_a, blk_b = blks[:_G], blks[_G:]

    def fire(g, bufs, sem):
        idx16 = idx_v[pl.ds(g * _G, _G)]
        copies = []
        for j in range(_G):
            blk0 = pl.multiple_of((idx16[j] >> 7) << 7, 128)
            copies.append(
                pltpu.async_copy(tt_hbm.at[:, pl.ds(blk0, 128)], bufs[j],
                                 sem)
            )
        return idx16, copies

    def extract(g, idx16, bufs, copies):
        for j in range(_G):
            copies[j].wait()
            col = jnp.broadcast_to(idx16[j] & 127, (16,))
            vals = plsc.load_gather(bufs[j], [lanes, col])
            pos = jnp.broadcast_to(g * _G + j, (16,))
            plsc.store_scatter(out_v, [lanes, pos], vals)

    def pair(i, carry):
        ga = i * 2
        idx_a, cp_a = fire(ga, blk_a, sem_a)
        idx_b, cp_b = fire(ga + 1, blk_b, sem_b)
        extract(ga, idx_a, blk_a, cp_a)
        extract(ga + 1, idx_b, blk_b, cp_b)
        return carry

    lax.fori_loop(0, _NGRP // 2, pair, 0)

    pltpu.sync_copy(out_v, ot_hbm.at[:, pl.ds(wid * _BPW, _BPW)])


_mesh = plsc.VectorSubcoreMesh(core_axis_name="c", subcore_axis_name="s")

_gather_call = functools.partial(
    pl.kernel,
    out_type=jax.ShapeDtypeStruct((EMBEDDING_DIM, BATCH), jnp.float32),
    mesh=_mesh,
    scratch_types=[
        pltpu.VMEM((_BPW,), jnp.int32),
        pltpu.VMEM((EMBEDDING_DIM, _BPW), jnp.float32),
        pltpu.SemaphoreType.DMA,
        pltpu.SemaphoreType.DMA,
    ] + [pltpu.VMEM((EMBEDDING_DIM, 128), jnp.float32)
         for _ in range(2 * _G)],
    compiler_params=pltpu.CompilerParams(needs_layout_passes=False),
)(_gather_body)


def kernel(indices, table):
    idx = jnp.asarray(indices, jnp.int32)
    return _gather_call(table.T, idx).T


# speedup vs baseline: 5.7614x; 1.0234x over previous
import functools

import jax
import jax.numpy as jnp
from jax import lax
from jax.experimental import pallas as pl
from jax.experimental.pallas import tpu as pltpu
from jax.experimental.pallas import tpu_sc as plsc

NUM_EMBEDDINGS = 1000000
EMBEDDING_DIM = 16
BATCH = 16384

_NC = 2
_NS = 16
_NW = _NC * _NS
_BPW = BATCH // _NW
_G = 16
_NGRP = _BPW // _G


def _gather_body(tt_hbm, idx_hbm, ot_hbm, idx_v, out_v, blk_a, blk_b,
                 sem_a, sem_b):
    wid = lax.axis_index("s") * _NC + lax.axis_index("c")
    pltpu.sync_copy(idx_hbm.at[pl.ds(wid * _BPW, _BPW)], idx_v)

    lanes = lax.iota(jnp.int32, 16)

    def fire(g, buf, sem):
        idx16 = idx_v[pl.ds(g * _G, _G)]
        copies = []
        for j in range(_G):
            blk0 = pl.multiple_of((idx16[j] >> 7) << 7, 128)
            copies.append(
                pltpu.async_copy(
                    tt_hbm.at[:, pl.ds(blk0, 128)],
                    buf.at[:, pl.ds(j * 128, 128)],
                    sem,
                )
            )
        return idx16, copies

    def extract(g, idx16, buf, copies):
        for c in copies:
            c.wait()
        colv = (idx16 & 127) + lanes * 128
        posv = g * _G + lanes
        for d in range(EMBEDDING_DIM):
            dv = jnp.broadcast_to(d, (16,))
            vals = plsc.load_gather(buf, [dv, colv])
            plsc.store_scatter(out_v, [dv, posv], vals)

    def pair(i, carry):
        ga = i * 2
        idx_a, cp_a = fire(ga, blk_a, sem_a)
        idx_b, cp_b = fire(ga + 1, blk_b, sem_b)
        extract(ga, idx_a, blk_a, cp_a)
        extract(ga + 1, idx_b, blk_b, cp_b)
        return carry

    lax.fori_loop(0, _NGRP // 2, pair, 0)

    pltpu.sync_copy(out_v, ot_hbm.at[:, pl.ds(wid * _BPW, _BPW)])


_mesh = plsc.VectorSubcoreMesh(core_axis_name="c", subcore_axis_name="s")

_gather_call = functools.partial(
    pl.kernel,
    out_type=jax.ShapeDtypeStruct((EMBEDDING_DIM, BATCH), jnp.float32),
    mesh=_mesh,
    scratch_types=[
        pltpu.VMEM((_BPW,), jnp.int32),
        pltpu.VMEM((EMBEDDING_DIM, _BPW), jnp.float32),
        pltpu.VMEM((EMBEDDING_DIM, _G * 128), jnp.float32),
        pltpu.VMEM((EMBEDDING_DIM, _G * 128), jnp.float32),
        pltpu.SemaphoreType.DMA,
        pltpu.SemaphoreType.DMA,
    ],
    compiler_params=pltpu.CompilerParams(needs_layout_passes=False),
)(_gather_body)


def kernel(indices, table):
    idx = jnp.asarray(indices, jnp.int32)
    return _gather_call(table.T, idx).T
